# baseline (speedup 1.0000x reference)
import functools

import jax
import jax.numpy as jnp
from jax.experimental import pallas as pl
from jax.experimental.pallas import tpu as pltpu

_BN = 512


def _cd_body(x1c_ref, x2r_ref, d1_ref, d2_ref):
    nb = pl.program_id(1)
    x1c = x1c_ref[0]
    x2r = x2r_ref[0]
    d = x1c[:, 3:4] + x2r[3:4, :]
    d = d + x1c[:, 0:1] * x2r[0:1, :]
    d = d + x1c[:, 1:2] * x2r[1:2, :]
    d = d + x1c[:, 2:3] * x2r[2:3, :]
    d1_ref[0] = jnp.min(d, axis=1, keepdims=True)
    part = jnp.min(d, axis=0, keepdims=True)

    @pl.when(nb == 0)
    def _():
        d2_ref[0] = part

    @pl.when(nb > 0)
    def _():
        d2_ref[0] = jnp.minimum(d2_ref[0], part)


@jax.jit
def kernel(xyz1, xyz2):
    B, N, _ = xyz1.shape
    M = xyz2.shape[1]
    x1s = jnp.sum(xyz1 * xyz1, axis=-1, keepdims=True)
    x2s = jnp.sum(xyz2 * xyz2, axis=-1, keepdims=True)
    x1c = jnp.concatenate([-2.0 * xyz1, x1s], axis=-1)
    x2r = jnp.transpose(
        jnp.concatenate([xyz2, x2s], axis=-1), (0, 2, 1))
    grid = (B, N // _BN)
    d1, d2 = pl.pallas_call(
        _cd_body,
        grid=grid,
        in_specs=[
            pl.BlockSpec((1, _BN, 4), lambda b, i: (b, i, 0)),
            pl.BlockSpec((1, 4, M), lambda b, i: (b, 0, 0)),
        ],
        out_specs=[
            pl.BlockSpec((1, _BN, 1), lambda b, i: (b, i, 0)),
            pl.BlockSpec((1, 1, M), lambda b, i: (b, 0, 0)),
        ],
        out_shape=[
            jax.ShapeDtypeStruct((B, N, 1), jnp.float32),
            jax.ShapeDtypeStruct((B, 1, M), jnp.float32),
        ],
        compiler_params=pltpu.CompilerParams(
            dimension_semantics=("parallel", "arbitrary")),
    )(x1c, x2r)
    return d1.reshape(B, N), d2.reshape(B, M)


# speedup vs baseline: 2.0653x; 2.0653x over previous
import functools

import jax
import jax.numpy as jnp
from jax.experimental import pallas as pl
from jax.experimental.pallas import tpu as pltpu

_BN = 512


def _cd_body(x1_ref, x2t_ref, d1_ref, d2_ref):
    nb = pl.program_id(1)
    x1 = x1_ref[0]
    x2t = x2t_ref[0]
    d = jax.lax.dot_general(
        x1, x2t, dimension_numbers=(((1,), (0,)), ((), ())),
        preferred_element_type=jnp.float32)
    d1_ref[0] = jnp.min(d, axis=1, keepdims=True)
    part = jnp.min(d, axis=0, keepdims=True)

    @pl.when(nb == 0)
    def _():
        d2_ref[0] = part

    @pl.when(nb > 0)
    def _():
        d2_ref[0] = jnp.minimum(d2_ref[0], part)


@jax.jit
def kernel(xyz1, xyz2):
    B, N, _ = xyz1.shape
    M = xyz2.shape[1]
    x1s = jnp.sum(xyz1 * xyz1, axis=-1, keepdims=True)
    x2s = jnp.sum(xyz2 * xyz2, axis=-1, keepdims=True)
    ones1 = jnp.ones((B, N, 1), jnp.float32)
    ones2 = jnp.ones((B, M, 1), jnp.float32)
    x1a = jnp.concatenate([-2.0 * xyz1, x1s, ones1], axis=-1)
    x2a = jnp.transpose(
        jnp.concatenate([xyz2, ones2, x2s], axis=-1), (0, 2, 1))
    grid = (B, N // _BN)
    d1, d2 = pl.pallas_call(
        _cd_body,
        grid=grid,
        in_specs=[
            pl.BlockSpec((1, _BN, 5), lambda b, i: (b, i, 0)),
            pl.BlockSpec((1, 5, M), lambda b, i: (b, 0, 0)),
        ],
        out_specs=[
            pl.BlockSpec((1, _BN, 1), lambda b, i: (b, i, 0)),
            pl.BlockSpec((1, 1, M), lambda b, i: (b, 0, 0)),
        ],
        out_shape=[
            jax.ShapeDtypeStruct((B, N, 1), jnp.float32),
            jax.ShapeDtypeStruct((B, 1, M), jnp.float32),
        ],
        compiler_params=pltpu.CompilerParams(
            dimension_semantics=("parallel", "arbitrary")),
    )(x1a, x2a)
    return d1.reshape(B, N), d2.reshape(B, M)
